# Initial kernel scaffold; baseline (speedup 1.0000x reference)
import jax
import jax.numpy as jnp
from jax.experimental import pallas as pl


def kernel(token_ids, encoder_context, emb_table):
    raise NotImplementedError("write your pallas kernel here")


# speedup vs baseline: 1.4423x; 1.4423x over previous
import functools

import jax
import jax.numpy as jnp
from jax import lax
from jax.experimental import pallas as pl
from jax.experimental.pallas import tpu as pltpu
from jax.experimental.pallas import tpu_sc as plsc

N_EMBD = 1024
NUM_CORES = 2
NUM_SUBCORES = 16
NUM_WORKERS = NUM_CORES * NUM_SUBCORES
CHUNK = 32
NBUF = 2


def _emb_body(table_hbm, idx_hbm, out_hbm, idx_v, bufs, sem0, sem1):
    nchunk = idx_hbm.shape[1]
    sems = (sem0, sem1)
    wid = lax.axis_index("s") * NUM_CORES + lax.axis_index("c")
    base = wid * (nchunk * CHUNK)

    pltpu.sync_copy(idx_hbm.at[wid], idx_v)

    pltpu.async_copy(table_hbm.at[idx_v.at[0]], bufs.at[0], sems[0])

    def outer(g):
        for b in range(NBUF):
            cur = g * NBUF + b
            nxt = cur + 1

            @pl.when(nxt < nchunk)
            def _():
                pltpu.async_copy(
                    table_hbm.at[idx_v.at[nxt]], bufs.at[1 - b], sems[1 - b]
                )

            pltpu.make_async_copy(
                table_hbm.at[idx_v.at[cur]], bufs.at[b], sems[b]
            ).wait()

            pltpu.sync_copy(
                bufs.at[b], out_hbm.at[pl.ds(base + cur * CHUNK, CHUNK)]
            )

    pl.loop(0, nchunk // NBUF)(outer)


@functools.partial(jax.jit, static_argnums=(2, 3))
def _sc_embedding_lookup(emb_table, idx, n_ids, nchunk):
    grid_kernel = pl.kernel(
        _emb_body,
        out_type=jax.ShapeDtypeStruct((n_ids, N_EMBD), jnp.float32),
        mesh=plsc.VectorSubcoreMesh(
            core_axis_name="c",
            subcore_axis_name="s",
            num_cores=NUM_CORES,
            num_subcores=NUM_SUBCORES,
        ),
        scratch_types=[
            pltpu.VMEM((nchunk, CHUNK), jnp.int32),
            pltpu.VMEM((NBUF, CHUNK, N_EMBD), jnp.float32),
            pltpu.SemaphoreType.DMA,
            pltpu.SemaphoreType.DMA,
        ],
    )
    return grid_kernel(emb_table, idx)


def kernel(token_ids, encoder_context, emb_table):
    batch, seq_len = token_ids.shape
    n_ids = batch * seq_len
    nchunk = n_ids // (NUM_WORKERS * CHUNK)
    idx = token_ids.astype(jnp.int32).reshape(NUM_WORKERS, nchunk, CHUNK)
    flat = _sc_embedding_lookup(emb_table, idx, n_ids, nchunk)
    return (flat.reshape(batch, seq_len, N_EMBD), encoder_context)


# speedup vs baseline: 1.4657x; 1.0162x over previous
import functools

import jax
import jax.numpy as jnp
from jax import lax
from jax.experimental import pallas as pl
from jax.experimental.pallas import tpu as pltpu
from jax.experimental.pallas import tpu_sc as plsc

N_EMBD = 1024
NUM_CORES = 2
NUM_SUBCORES = 16
NUM_WORKERS = NUM_CORES * NUM_SUBCORES
CHUNK = 32
NBUF = 2


def _emb_body(table_hbm, idx_hbm, out_hbm, idx_v, bufs, sem0, sem1):
    nchunk = idx_hbm.shape[1]
    sems = (sem0, sem1)
    wid = lax.axis_index("s") * NUM_CORES + lax.axis_index("c")
    base = wid * (nchunk * CHUNK)

    pltpu.sync_copy(idx_hbm.at[wid], idx_v)

    pltpu.async_copy(table_hbm.at[idx_v.at[0]], bufs.at[0], sems[0])

    def outer(g):
        for b in range(NBUF):
            cur = g * NBUF + b
            nxt = cur + 1

            @pl.when(nxt < nchunk)
            def _():
                pltpu.async_copy(
                    table_hbm.at[idx_v.at[nxt]], bufs.at[1 - b], sems[1 - b]
                )

            pltpu.make_async_copy(
                table_hbm.at[idx_v.at[cur]], bufs.at[b], sems[b]
            ).wait()

            pltpu.sync_copy(
                bufs.at[b], out_hbm.at[pl.ds(base + cur * CHUNK, CHUNK)]
            )

    pl.loop(0, nchunk // NBUF)(outer)


def _copy_body(src_ref, dst_ref):
    dst_ref[...] = src_ref[...]


def _tc_copy(x):
    rows, cols = x.shape
    blk = 512
    return pl.pallas_call(
        _copy_body,
        grid=(rows // blk,),
        in_specs=[pl.BlockSpec((blk, cols), lambda i: (i, 0))],
        out_specs=pl.BlockSpec((blk, cols), lambda i: (i, 0)),
        out_shape=jax.ShapeDtypeStruct((rows, cols), x.dtype),
    )(x)


@functools.partial(jax.jit, static_argnums=(2, 3))
def _sc_embedding_lookup(emb_table, idx, n_ids, nchunk):
    grid_kernel = pl.kernel(
        _emb_body,
        out_type=jax.ShapeDtypeStruct((n_ids, N_EMBD), jnp.float32),
        mesh=plsc.VectorSubcoreMesh(
            core_axis_name="c",
            subcore_axis_name="s",
            num_cores=NUM_CORES,
            num_subcores=NUM_SUBCORES,
        ),
        scratch_types=[
            pltpu.VMEM((nchunk, CHUNK), jnp.int32),
            pltpu.VMEM((NBUF, CHUNK, N_EMBD), jnp.float32),
            pltpu.SemaphoreType.DMA,
            pltpu.SemaphoreType.DMA,
        ],
    )
    return grid_kernel(emb_table, idx)


def kernel(token_ids, encoder_context, emb_table):
    batch, seq_len = token_ids.shape
    n_ids = batch * seq_len
    nchunk = n_ids // (NUM_WORKERS * CHUNK)
    idx = token_ids.astype(jnp.int32).reshape(NUM_WORKERS, nchunk, CHUNK)
    flat = _sc_embedding_lookup(emb_table, idx, n_ids, nchunk)
    ctx = _tc_copy(encoder_context.reshape(n_ids, N_EMBD))
    return (flat.reshape(batch, seq_len, N_EMBD), ctx.reshape(batch, seq_len, N_EMBD))
